# baseline (speedup 1.0000x reference)
import jax
import jax.numpy as jnp
from jax import lax
from jax.experimental import pallas as pl
from jax.experimental.pallas import tpu as pltpu
from jax.experimental.pallas import tpu_sc as plsc

SEG = 256
L = 128
NPTS = 20000
NROWS_IN = 160
TOPK = 6000
CPAD = 6144
NBLK = 48
NBOX = 56
NOUT = 1000
NMS_T = 0.7
BIG = 1 << 26
NC, NS = 2, 16
NW = NC * NS


def _bitonic(key, val, nrows, win_fn, seg_rows=None):
    rowi = lax.broadcasted_iota(jnp.int32, (nrows, L), 0)
    lanei = lax.broadcasted_iota(jnp.int32, (nrows, L), 1)
    if seg_rows is None:
        seg_rows = nrows
    rowb = rowi & (seg_rows - 1)
    n = seg_rows * L
    k = 2
    while k <= n:
        j = k // 2
        while j >= 1:
            if j >= L:
                jr = j // L
                km = jnp.roll(key, -jr, axis=0)
                kp = jnp.roll(key, jr, axis=0)
                vm = jnp.roll(val, -jr, axis=0)
                vp = jnp.roll(val, jr, axis=0)
                lower = (rowb & jr) == 0
            else:
                km = pltpu.roll(key, L - j, 1)
                kp = pltpu.roll(key, j, 1)
                vm = pltpu.roll(val, L - j, 1)
                vp = pltpu.roll(val, j, 1)
                lower = (lanei & j) == 0
            pkey = jnp.where(lower, km, kp)
            pval = jnp.where(lower, vm, vp)
            if k >= L:
                asc = (rowb & (k // L)) == 0
            else:
                asc = (lanei & k) == 0
            win = win_fn(key, pkey, val, pval)
            takex = ~((asc == lower) ^ win)
            key = jnp.where(takex, key, pkey)
            val = jnp.where(takex, val, pval)
            j //= 2
        k *= 2
    return key, val


def _sort_body(s_ref, idx_ref):
    pad = jnp.full((SEG - NROWS_IN, L), -1.0, jnp.float32)
    key = lax.bitcast_convert_type(
        jnp.concatenate([s_ref[0], pad], axis=0), jnp.int32)
    rowi = lax.broadcasted_iota(jnp.int32, (SEG, L), 0)
    lanei = lax.broadcasted_iota(jnp.int32, (SEG, L), 1)
    idx = rowi * L + lanei

    def win(k, pk, v, pv):
        return (k > pk) | ((k == pk) & (v < pv))

    _, idx = _bitonic(key, idx, SEG, win)
    idx_ref[0] = idx[:NBLK]


def _gather_planes_body(anch_hbm, bbox_hbm, idx_hbm, out_hbm, idx_v, gidx_v,
                        vals_v, sem):
    wid = lax.axis_index("s") * NC + lax.axis_index("c")
    b = wid // 8
    p = wid % 8
    c = p % 4
    base = b * (NPTS * 4) + c
    pltpu.sync_copy(idx_hbm.at[b], idx_v)

    def mk(r, _):
        for s in range(8):
            iv = idx_v[r, pl.ds(s * 16, 16)]
            gidx_v[r, pl.ds(s * 16, 16)] = iv * 4 + base
        return 0

    lax.fori_loop(0, NBLK, mk, 0)

    @pl.when(p < 4)
    def _():
        copies = [
            pltpu.async_copy(anch_hbm.at[gidx_v.at[i]], vals_v.at[i], sem)
            for i in range(NBLK)
        ]
        for cp in copies:
            cp.wait()

    @pl.when(p >= 4)
    def _():
        copies = [
            pltpu.async_copy(bbox_hbm.at[gidx_v.at[i]], vals_v.at[i], sem)
            for i in range(NBLK)
        ]
        for cp in copies:
            cp.wait()

    pltpu.sync_copy(vals_v, out_hbm.at[wid])


def _iou_gt(y1i, x1i, y2i, x2i, ai, y1j, x1j, y2j, x2j, aj):
    yy1 = jnp.maximum(y1i, y1j)
    xx1 = jnp.maximum(x1i, x1j)
    yy2 = jnp.minimum(y2i, y2j)
    xx2 = jnp.minimum(x2i, x2j)
    inter = jnp.maximum(yy2 - yy1, 0.0) * jnp.maximum(xx2 - xx1, 0.0)
    return inter > NMS_T * (ai + aj - inter)


def _nms_body(g_ref, keep_ref, box_ref, sel_ref):
    g = g_ref[0]
    ay1, ax1, ay2, ax2 = g[0], g[1], g[2], g[3]
    dy, dx, dh, dw = g[4], g[5], g[6], g[7]
    ha = ay2 - ay1
    wa = ax2 - ax1
    cy = ay1 + 0.5 * ha + dy * 0.1 * ha
    cx = ax1 + 0.5 * wa + dx * 0.1 * wa
    h = ha * jnp.exp(dh * 0.2)
    w = wa * jnp.exp(dw * 0.2)
    y1 = jnp.clip(cy - 0.5 * h, 0.0, 1.0)
    x1 = jnp.clip(cx - 0.5 * w, 0.0, 1.0)
    y2 = jnp.clip(cy - 0.5 * h + h, 0.0, 1.0)
    x2 = jnp.clip(cx - 0.5 * w + w, 0.0, 1.0)
    zpad = jnp.zeros((NBOX - NBLK, L), jnp.float32)
    box_ref[0, 0] = jnp.concatenate([y1, zpad], axis=0)
    box_ref[0, 1] = jnp.concatenate([x1, zpad], axis=0)
    box_ref[0, 2] = jnp.concatenate([y2, zpad], axis=0)
    box_ref[0, 3] = jnp.concatenate([x2, zpad], axis=0)
    keep_ref[0] = jnp.zeros((NBLK, L), jnp.float32)

    li = lax.broadcasted_iota(jnp.int32, (L, L), 0)
    lj = lax.broadcasted_iota(jnp.int32, (L, L), 1)
    ltri = jnp.where(lj < li, 1.0, 0.0)
    lane1 = lax.broadcasted_iota(jnp.int32, (1, L), 1)

    def row(c, t):
        return box_ref[0, c, t][None]

    def block_body(state):
        t, cnt = state
        by1 = row(0, t)
        bx1 = row(1, t)
        by2 = row(2, t)
        bx2 = row(3, t)
        ba = (by2 - by1) * (bx2 - bx1)
        bval = jnp.where(t * L + lane1 < TOPK, 1.0, 0.0)
        packed = jnp.concatenate(
            [by1, bx1, by2, bx2, ba, bval, jnp.zeros((2, L), jnp.float32)],
            axis=0)
        tp = lax.transpose(packed, (1, 0))
        ci = [tp[:, i:i + 1] for i in range(5)]
        cval = tp[:, 5:6]

        def cross(s, acc):
            ry1 = row(0, s)
            rx1 = row(1, s)
            ry2 = row(2, s)
            rx2 = row(3, s)
            ra = (ry2 - ry1) * (rx2 - rx1)
            rk = keep_ref[0, s][None]
            m = _iou_gt(*ci, ry1, rx1, ry2, rx2, ra) & (rk > 0.5)
            return jnp.maximum(
                acc, jnp.max(jnp.where(m, 1.0, 0.0), axis=1, keepdims=True))

        sup0 = lax.fori_loop(0, t, cross, jnp.zeros((L, 1), jnp.float32))

        M = jnp.where(_iou_gt(*ci, by1, bx1, by2, bx2, ba), 1.0, 0.0) * ltri
        init = (1.0 - sup0) * cval

        def fp_cond(st):
            it, ch, _ = st
            return (ch > 0.5) & (it < L)

        def fp_body(st):
            it, _, kc = st
            sup = jnp.dot(M, kc, preferred_element_type=jnp.float32)
            nk = init * jnp.where(sup > 0.5, 0.0, 1.0)
            return it + 1, jnp.max(jnp.abs(nk - kc)), nk

        _, _, kc = lax.while_loop(fp_cond, fp_body, (0, 1.0, init))

        kr = lax.transpose(jnp.broadcast_to(kc, (L, 8)), (1, 0))[0]
        keep_ref[0, t] = kr
        return t + 1, cnt + jnp.sum(kc).astype(jnp.int32)

    def block_cond(state):
        t, cnt = state
        return (t < NBLK) & (cnt < NOUT)

    lax.while_loop(block_cond, block_body, (0, jnp.int32(0)))

    K = keep_ref[0]
    lt_inc = jnp.where(li <= lj, 1.0, 0.0)
    incl_row = jnp.dot(K, lt_inc, preferred_element_type=jnp.float32)
    rowsum = jnp.dot(K, jnp.ones((L, 1), jnp.float32),
                     preferred_element_type=jnp.float32)
    ri = lax.broadcasted_iota(jnp.int32, (NBLK, NBLK), 0)
    rj = lax.broadcasted_iota(jnp.int32, (NBLK, NBLK), 1)
    t48 = jnp.where(rj < ri, 1.0, 0.0)
    off = jnp.dot(t48, rowsum, preferred_element_type=jnp.float32)
    pos = (incl_row + off - 1.0).astype(jnp.int32)

    rowi = lax.broadcasted_iota(jnp.int32, (NBLK, L), 0)
    lanei = lax.broadcasted_iota(jnp.int32, (NBLK, L), 1)
    srci = rowi * L + lanei
    kept = K > 0.5
    skey = jnp.concatenate(
        [jnp.where(kept, pos, BIG), jnp.full((64 - NBLK, L), BIG, jnp.int32)],
        axis=0)
    sval = jnp.concatenate(
        [jnp.where(kept, srci, NBLK * L),
         jnp.full((64 - NBLK, L), NBLK * L, jnp.int32)], axis=0)

    def win(k, pk, v, pv):
        return (k < pk) | ((k == pk) & (v < pv))

    _, sval = _bitonic(skey, sval, 64, win)
    sel_ref[0] = sval[:8]


def _gather_out_body(box_hbm, sel_hbm, out_hbm, sel_v, gidx_v, vals_v, sem):
    wid = lax.axis_index("s") * NC + lax.axis_index("c")
    nb = sel_hbm.shape[0]

    @pl.when(wid < nb * 4)
    def _():
        b = wid // 4
        p = wid % 4
        base = (b * 4 + p) * (NBOX * L)
        pltpu.sync_copy(sel_hbm.at[b], sel_v)

        def mk(r, _):
            for s in range(8):
                sv = sel_v[r, pl.ds(s * 16, 16)]
                gidx_v[r, pl.ds(s * 16, 16)] = sv + base
            return 0

        lax.fori_loop(0, 8, mk, 0)
        copies = [
            pltpu.async_copy(box_hbm.at[gidx_v.at[i]], vals_v.at[i], sem)
            for i in range(8)
        ]
        for cp in copies:
            cp.wait()
        pltpu.sync_copy(vals_v, out_hbm.at[wid])


def kernel(rpn_probs, rpn_bbox, anchors):
    nb = rpn_probs.shape[0]
    scores = rpn_probs[:, :, 1]
    sp = jnp.pad(scores, ((0, 0), (0, NROWS_IN * L - NPTS)),
                 constant_values=-1.0).reshape(nb, NROWS_IN, L)

    idx_sorted = pl.pallas_call(
        _sort_body,
        grid=(nb,),
        in_specs=[pl.BlockSpec((1, NROWS_IN, L), lambda b: (b, 0, 0))],
        out_specs=pl.BlockSpec((1, NBLK, L), lambda b: (b, 0, 0)),
        out_shape=jax.ShapeDtypeStruct((nb, NBLK, L), jnp.int32),
    )(sp)

    gathered = pl.kernel(
        _gather_planes_body,
        out_type=jax.ShapeDtypeStruct((nb * 8, NBLK, L), jnp.float32),
        mesh=plsc.VectorSubcoreMesh(core_axis_name="c", subcore_axis_name="s",
                                    num_cores=NC, num_subcores=NS),
        scratch_types=[
            pltpu.VMEM((NBLK, L), jnp.int32),
            pltpu.VMEM((NBLK, L), jnp.int32),
            pltpu.VMEM((NBLK, L), jnp.float32),
            pltpu.SemaphoreType.DMA,
        ],
    )(anchors.reshape(-1), rpn_bbox.reshape(-1), idx_sorted)
    planes = gathered.reshape(nb, 8, NBLK, L)

    keep, boxes, sel = pl.pallas_call(
        _nms_body,
        grid=(nb,),
        in_specs=[pl.BlockSpec((1, 8, NBLK, L), lambda b: (b, 0, 0, 0))],
        out_specs=[
            pl.BlockSpec((1, NBLK, L), lambda b: (b, 0, 0)),
            pl.BlockSpec((1, 4, NBOX, L), lambda b: (b, 0, 0, 0)),
            pl.BlockSpec((1, 8, L), lambda b: (b, 0, 0)),
        ],
        out_shape=[
            jax.ShapeDtypeStruct((nb, NBLK, L), jnp.float32),
            jax.ShapeDtypeStruct((nb, 4, NBOX, L), jnp.float32),
            jax.ShapeDtypeStruct((nb, 8, L), jnp.int32),
        ],
    )(planes)

    picked = pl.kernel(
        _gather_out_body,
        out_type=jax.ShapeDtypeStruct((nb * 4, 8, L), jnp.float32),
        mesh=plsc.VectorSubcoreMesh(core_axis_name="c", subcore_axis_name="s",
                                    num_cores=NC, num_subcores=NS),
        scratch_types=[
            pltpu.VMEM((8, L), jnp.int32),
            pltpu.VMEM((8, L), jnp.int32),
            pltpu.VMEM((8, L), jnp.float32),
            pltpu.SemaphoreType.DMA,
        ],
    )(boxes.reshape(-1), sel)

    out = picked.reshape(nb, 4, 8 * L)[:, :, :NOUT]
    return out.transpose(0, 2, 1)


# speedup vs baseline: 1.4541x; 1.4541x over previous
import jax
import jax.numpy as jnp
from jax import lax
from jax.experimental import pallas as pl
from jax.experimental.pallas import tpu as pltpu
from jax.experimental.pallas import tpu_sc as plsc

R = 256
L = 128
NPTS = 20000
NROWS_IN = 160
TOPK = 6000
CPAD = 6144
NBLK = 48
NBOX = 56
NOUT = 1000
NMS_T = 0.7
BIG = 1 << 26
NC, NS = 2, 16
NW = NC * NS


def _bitonic(key, val, nrows, win_fn):
    rowi = lax.broadcasted_iota(jnp.int32, (nrows, L), 0)
    lanei = lax.broadcasted_iota(jnp.int32, (nrows, L), 1)
    n = nrows * L
    k = 2
    while k <= n:
        j = k // 2
        while j >= 1:
            if j >= L:
                jr = j // L
                km = jnp.roll(key, -jr, axis=0)
                kp = jnp.roll(key, jr, axis=0)
                vm = jnp.roll(val, -jr, axis=0)
                vp = jnp.roll(val, jr, axis=0)
                lower = (rowi & jr) == 0
            else:
                km = pltpu.roll(key, L - j, 1)
                kp = pltpu.roll(key, j, 1)
                vm = pltpu.roll(val, L - j, 1)
                vp = pltpu.roll(val, j, 1)
                lower = (lanei & j) == 0
            pkey = jnp.where(lower, km, kp)
            pval = jnp.where(lower, vm, vp)
            if k >= L:
                asc = (rowi & (k // L)) == 0
            else:
                asc = (lanei & k) == 0
            win = win_fn(key, pkey, val, pval)
            takex = ~((asc == lower) ^ win)
            key = jnp.where(takex, key, pkey)
            val = jnp.where(takex, val, pval)
            j //= 2
        k *= 2
    return key, val


def _sort_body(s_ref, idx_ref):
    s = s_ref[0]
    key = lax.bitcast_convert_type(
        jnp.concatenate([s, jnp.full((R - NROWS_IN, L), -1.0, jnp.float32)],
                        axis=0),
        jnp.int32)
    rowi = lax.broadcasted_iota(jnp.int32, (R, L), 0)
    lanei = lax.broadcasted_iota(jnp.int32, (R, L), 1)
    idx = rowi * L + lanei

    def win(k, pk, v, pv):
        return (k > pk) | ((k == pk) & (v < pv))

    _, idx = _bitonic(key, idx, R, win)
    idx_ref[0] = idx[:NBLK]


def _make_gather_body(njobs, nrows):
    def body(flat_hbm, idx_hbm, out_hbm, idx_v, vals_v, sem):
        wid = lax.axis_index("s") * NC + lax.axis_index("c")

        @pl.when(wid < njobs)
        def _():
            pltpu.sync_copy(idx_hbm.at[wid], idx_v)
            copies = [
                pltpu.async_copy(flat_hbm.at[idx_v.at[i]], vals_v.at[i], sem)
                for i in range(nrows)
            ]
            for c in copies:
                c.wait()
            pltpu.sync_copy(vals_v, out_hbm.at[wid])

    return body


def _sc_gather(flat, idx, njobs, nrows):
    return pl.kernel(
        _make_gather_body(njobs, nrows),
        out_type=jax.ShapeDtypeStruct((njobs, nrows, L), jnp.float32),
        mesh=plsc.VectorSubcoreMesh(core_axis_name="c", subcore_axis_name="s",
                                    num_cores=NC, num_subcores=NS),
        scratch_types=[
            pltpu.VMEM((nrows, L), jnp.int32),
            pltpu.VMEM((nrows, L), jnp.float32),
            pltpu.SemaphoreType.DMA,
        ],
    )(flat, idx)


def _iou_gt(y1i, x1i, y2i, x2i, ai, y1j, x1j, y2j, x2j, aj):
    yy1 = jnp.maximum(y1i, y1j)
    xx1 = jnp.maximum(x1i, x1j)
    yy2 = jnp.minimum(y2i, y2j)
    xx2 = jnp.minimum(x2i, x2j)
    inter = jnp.maximum(yy2 - yy1, 0.0) * jnp.maximum(xx2 - xx1, 0.0)
    return inter > NMS_T * (ai + aj - inter)


def _nms_body(g_ref, keep_ref, box_ref, sel_ref):
    g = g_ref[0]
    ay1, ax1, ay2, ax2 = g[0], g[1], g[2], g[3]
    dy, dx, dh, dw = g[4], g[5], g[6], g[7]
    ha = ay2 - ay1
    wa = ax2 - ax1
    cy = ay1 + 0.5 * ha + dy * 0.1 * ha
    cx = ax1 + 0.5 * wa + dx * 0.1 * wa
    h = ha * jnp.exp(dh * 0.2)
    w = wa * jnp.exp(dw * 0.2)
    y1 = jnp.clip(cy - 0.5 * h, 0.0, 1.0)
    x1 = jnp.clip(cx - 0.5 * w, 0.0, 1.0)
    y2 = jnp.clip(cy - 0.5 * h + h, 0.0, 1.0)
    x2 = jnp.clip(cx - 0.5 * w + w, 0.0, 1.0)
    zpad = jnp.zeros((NBOX - NBLK, L), jnp.float32)
    box_ref[0, 0] = jnp.concatenate([y1, zpad], axis=0)
    box_ref[0, 1] = jnp.concatenate([x1, zpad], axis=0)
    box_ref[0, 2] = jnp.concatenate([y2, zpad], axis=0)
    box_ref[0, 3] = jnp.concatenate([x2, zpad], axis=0)
    keep_ref[0] = jnp.zeros((NBLK, L), jnp.float32)

    li = lax.broadcasted_iota(jnp.int32, (L, L), 0)
    lj = lax.broadcasted_iota(jnp.int32, (L, L), 1)
    ltri = jnp.where(lj < li, 1.0, 0.0)
    lane1 = lax.broadcasted_iota(jnp.int32, (1, L), 1)

    def row(c, t):
        return box_ref[0, c, t][None]

    def block_body(state):
        t, cnt = state
        by1 = row(0, t)
        bx1 = row(1, t)
        by2 = row(2, t)
        bx2 = row(3, t)
        ba = (by2 - by1) * (bx2 - bx1)
        bval = jnp.where(t * L + lane1 < TOPK, 1.0, 0.0)
        packed = jnp.concatenate(
            [by1, bx1, by2, bx2, ba, bval, jnp.zeros((2, L), jnp.float32)],
            axis=0)
        tp = lax.transpose(packed, (1, 0))
        ci = [tp[:, i:i + 1] for i in range(5)]
        cval = tp[:, 5:6]

        def cross(s, acc):
            ry1 = row(0, s)
            rx1 = row(1, s)
            ry2 = row(2, s)
            rx2 = row(3, s)
            ra = (ry2 - ry1) * (rx2 - rx1)
            rk = keep_ref[0, s][None]
            m = _iou_gt(*ci, ry1, rx1, ry2, rx2, ra) & (rk > 0.5)
            return jnp.maximum(
                acc, jnp.max(jnp.where(m, 1.0, 0.0), axis=1, keepdims=True))

        sup0 = lax.fori_loop(0, t, cross, jnp.zeros((L, 1), jnp.float32))

        M = jnp.where(_iou_gt(*ci, by1, bx1, by2, bx2, ba), 1.0, 0.0) * ltri
        init = (1.0 - sup0) * cval

        def fp_cond(st):
            it, ch, _ = st
            return (ch > 0.5) & (it < L)

        def fp_body(st):
            it, _, kc = st
            sup = jnp.dot(M, kc, preferred_element_type=jnp.float32)
            nk = init * jnp.where(sup > 0.5, 0.0, 1.0)
            return it + 1, jnp.max(jnp.abs(nk - kc)), nk

        _, _, kc = lax.while_loop(fp_cond, fp_body, (0, 1.0, init))

        kr = lax.transpose(jnp.broadcast_to(kc, (L, 8)), (1, 0))[0]
        keep_ref[0, t] = kr
        return t + 1, cnt + jnp.sum(kc).astype(jnp.int32)

    def block_cond(state):
        t, cnt = state
        return (t < NBLK) & (cnt < NOUT)

    lax.while_loop(block_cond, block_body, (0, jnp.int32(0)))

    K = keep_ref[0]
    lt_inc = jnp.where(li <= lj, 1.0, 0.0)
    incl_row = jnp.dot(K, lt_inc, preferred_element_type=jnp.float32)
    rowsum = jnp.dot(K, jnp.ones((L, 1), jnp.float32),
                     preferred_element_type=jnp.float32)
    ri = lax.broadcasted_iota(jnp.int32, (NBLK, NBLK), 0)
    rj = lax.broadcasted_iota(jnp.int32, (NBLK, NBLK), 1)
    t48 = jnp.where(rj < ri, 1.0, 0.0)
    off = jnp.dot(t48, rowsum, preferred_element_type=jnp.float32)
    pos = (incl_row + off - 1.0).astype(jnp.int32)

    rowi = lax.broadcasted_iota(jnp.int32, (NBLK, L), 0)
    lanei = lax.broadcasted_iota(jnp.int32, (NBLK, L), 1)
    srci = rowi * L + lanei
    kept = K > 0.5
    skey = jnp.concatenate(
        [jnp.where(kept, pos, BIG), jnp.full((64 - NBLK, L), BIG, jnp.int32)],
        axis=0)
    sval = jnp.concatenate(
        [jnp.where(kept, srci, NBLK * L),
         jnp.full((64 - NBLK, L), NBLK * L, jnp.int32)], axis=0)

    def win(k, pk, v, pv):
        return (k < pk) | ((k == pk) & (v < pv))

    _, sval = _bitonic(skey, sval, 64, win)
    sel_ref[0] = sval[:8]


def kernel(rpn_probs, rpn_bbox, anchors):
    nb = rpn_probs.shape[0]
    scores = rpn_probs[:, :, 1]
    sp = jnp.pad(scores, ((0, 0), (0, NROWS_IN * L - NPTS)),
                 constant_values=-1.0).reshape(nb, NROWS_IN, L)

    idx_sorted = pl.pallas_call(
        _sort_body,
        grid=(nb,),
        in_specs=[pl.BlockSpec((1, NROWS_IN, L), lambda b: (b, 0, 0))],
        out_specs=pl.BlockSpec((1, NBLK, L), lambda b: (b, 0, 0)),
        out_shape=jax.ShapeDtypeStruct((nb, NBLK, L), jnp.int32),
    )(sp)

    PPAD = 20096
    planes_src = jnp.concatenate(
        [anchors.transpose(0, 2, 1), rpn_bbox.transpose(0, 2, 1)], axis=1)
    flat = jnp.pad(planes_src, ((0, 0), (0, 0), (0, PPAD - NPTS))).reshape(-1)
    plane_off = ((jnp.arange(nb, dtype=jnp.int32) * 8)[:, None]
                 + jnp.arange(8, dtype=jnp.int32)[None, :]) * PPAD
    idx_g = (idx_sorted.reshape(nb, 1, CPAD)
             + plane_off[:, :, None]).reshape(nb * 8, NBLK, L)

    gathered = _sc_gather(flat, idx_g, nb * 8, NBLK)
    planes = gathered.reshape(nb, 8, NBLK, L)

    keep, boxes, sel = pl.pallas_call(
        _nms_body,
        grid=(nb,),
        in_specs=[pl.BlockSpec((1, 8, NBLK, L), lambda b: (b, 0, 0, 0))],
        out_specs=[
            pl.BlockSpec((1, NBLK, L), lambda b: (b, 0, 0)),
            pl.BlockSpec((1, 4, NBOX, L), lambda b: (b, 0, 0, 0)),
            pl.BlockSpec((1, 8, L), lambda b: (b, 0, 0)),
        ],
        out_shape=[
            jax.ShapeDtypeStruct((nb, NBLK, L), jnp.float32),
            jax.ShapeDtypeStruct((nb, 4, NBOX, L), jnp.float32),
            jax.ShapeDtypeStruct((nb, 8, L), jnp.int32),
        ],
    )(planes)

    box_flat = boxes.reshape(-1)
    sel_off = ((jnp.arange(nb, dtype=jnp.int32) * 4)[:, None]
               + jnp.arange(4, dtype=jnp.int32)[None, :]) * (NBOX * L)
    sel_g = (sel.reshape(nb, 1, 8 * L)
             + sel_off[:, :, None]).reshape(nb * 4, 8, L)

    picked = _sc_gather(box_flat, sel_g, nb * 4, 8)
    out = picked.reshape(nb, 4, 8 * L)[:, :, :NOUT]
    return out.transpose(0, 2, 1)


# speedup vs baseline: 1.5309x; 1.0528x over previous
import jax
import jax.numpy as jnp
from jax import lax
from jax.experimental import pallas as pl
from jax.experimental.pallas import tpu as pltpu
from jax.experimental.pallas import tpu_sc as plsc

R = 256
L = 128
NPTS = 20000
NROWS_IN = 160
TOPK = 6000
CPAD = 6144
NBLK = 48
NBOX = 56
NOUT = 1000
NMS_T = 0.7
BIG = 1 << 26
NC, NS = 2, 16
NW = NC * NS


def _bitonic(key, val, nrows, win_fn):
    rowi = lax.broadcasted_iota(jnp.int32, (nrows, L), 0)
    lanei = lax.broadcasted_iota(jnp.int32, (nrows, L), 1)
    n = nrows * L
    k = 2
    while k <= n:
        j = k // 2
        while j >= 1:
            if j >= L:
                jr = j // L
                km = jnp.roll(key, -jr, axis=0)
                kp = jnp.roll(key, jr, axis=0)
                vm = jnp.roll(val, -jr, axis=0)
                vp = jnp.roll(val, jr, axis=0)
                lower = (rowi & jr) == 0
            else:
                km = pltpu.roll(key, L - j, 1)
                kp = pltpu.roll(key, j, 1)
                vm = pltpu.roll(val, L - j, 1)
                vp = pltpu.roll(val, j, 1)
                lower = (lanei & j) == 0
            pkey = jnp.where(lower, km, kp)
            pval = jnp.where(lower, vm, vp)
            if k >= L:
                asc = (rowi & (k // L)) == 0
            else:
                asc = (lanei & k) == 0
            win = win_fn(key, pkey, val, pval)
            takex = ~((asc == lower) ^ win)
            key = jnp.where(takex, key, pkey)
            val = jnp.where(takex, val, pval)
            j //= 2
        k *= 2
    return key, val


def _sort_body(s_ref, idx_ref):
    s = s_ref[0]
    key = lax.bitcast_convert_type(
        jnp.concatenate([s, jnp.full((R - NROWS_IN, L), -1.0, jnp.float32)],
                        axis=0),
        jnp.int32)
    rowi = lax.broadcasted_iota(jnp.int32, (R, L), 0)
    lanei = lax.broadcasted_iota(jnp.int32, (R, L), 1)
    idx = rowi * L + lanei

    def win(k, pk, v, pv):
        return (k > pk) | ((k == pk) & (v < pv))

    _, idx = _bitonic(key, idx, R, win)
    idx_ref[0] = idx[:NBLK]


def _make_gather_body(njobs, nrows):
    def body(flat_hbm, idx_hbm, out_hbm, idx_v, vals_v, sem):
        wid = lax.axis_index("s") * NC + lax.axis_index("c")

        @pl.when(wid < njobs)
        def _():
            pltpu.sync_copy(idx_hbm.at[wid], idx_v)
            copies = [
                pltpu.async_copy(flat_hbm.at[idx_v.at[i]], vals_v.at[i], sem)
                for i in range(nrows)
            ]
            for c in copies:
                c.wait()
            pltpu.sync_copy(vals_v, out_hbm.at[wid])

    return body


def _sc_gather(flat, idx, njobs, nrows):
    return pl.kernel(
        _make_gather_body(njobs, nrows),
        out_type=jax.ShapeDtypeStruct((njobs, nrows, L), jnp.float32),
        mesh=plsc.VectorSubcoreMesh(core_axis_name="c", subcore_axis_name="s",
                                    num_cores=NC, num_subcores=NS),
        scratch_types=[
            pltpu.VMEM((nrows, L), jnp.int32),
            pltpu.VMEM((nrows, L), jnp.float32),
            pltpu.SemaphoreType.DMA,
        ],
    )(flat, idx)


def _iou_gt(y1i, x1i, y2i, x2i, ai, y1j, x1j, y2j, x2j, aj):
    yy1 = jnp.maximum(y1i, y1j)
    xx1 = jnp.maximum(x1i, x1j)
    yy2 = jnp.minimum(y2i, y2j)
    xx2 = jnp.minimum(x2i, x2j)
    inter = jnp.maximum(yy2 - yy1, 0.0) * jnp.maximum(xx2 - xx1, 0.0)
    return inter > NMS_T * (ai + aj - inter)


def _bitonic_keyed(key, vals, nrows, win_fn):
    rowi = lax.broadcasted_iota(jnp.int32, (nrows, L), 0)
    lanei = lax.broadcasted_iota(jnp.int32, (nrows, L), 1)
    n = nrows * L
    k = 2
    while k <= n:
        j = k // 2
        while j >= 1:
            if j >= L:
                jr = j // L
                km = jnp.roll(key, -jr, axis=0)
                kp = jnp.roll(key, jr, axis=0)
                pv = [(jnp.roll(v, -jr, axis=0), jnp.roll(v, jr, axis=0))
                      for v in vals]
                lower = (rowi & jr) == 0
            else:
                km = pltpu.roll(key, L - j, 1)
                kp = pltpu.roll(key, j, 1)
                pv = [(pltpu.roll(v, L - j, 1), pltpu.roll(v, j, 1))
                      for v in vals]
                lower = (lanei & j) == 0
            pkey = jnp.where(lower, km, kp)
            pvals = [jnp.where(lower, a, b) for a, b in pv]
            if k >= L:
                asc = (rowi & (k // L)) == 0
            else:
                asc = (lanei & k) == 0
            win = win_fn(key, pkey)
            takex = ~((asc == lower) ^ win)
            key = jnp.where(takex, key, pkey)
            vals = [jnp.where(takex, v, p) for v, p in zip(vals, pvals)]
            j //= 2
        k *= 2
    return key, vals


def _nms_body(g_ref, out_ref, keep_ref, box_ref):
    g = g_ref[0]
    ay1, ax1, ay2, ax2 = g[0], g[1], g[2], g[3]
    dy, dx, dh, dw = g[4], g[5], g[6], g[7]
    ha = ay2 - ay1
    wa = ax2 - ax1
    cy = ay1 + 0.5 * ha + dy * 0.1 * ha
    cx = ax1 + 0.5 * wa + dx * 0.1 * wa
    h = ha * jnp.exp(dh * 0.2)
    w = wa * jnp.exp(dw * 0.2)
    y1 = jnp.clip(cy - 0.5 * h, 0.0, 1.0)
    x1 = jnp.clip(cx - 0.5 * w, 0.0, 1.0)
    y2 = jnp.clip(cy - 0.5 * h + h, 0.0, 1.0)
    x2 = jnp.clip(cx - 0.5 * w + w, 0.0, 1.0)
    box_ref[0, 0] = y1
    box_ref[0, 1] = x1
    box_ref[0, 2] = y2
    box_ref[0, 3] = x2
    keep_ref[0] = jnp.zeros((NBLK, L), jnp.float32)

    li = lax.broadcasted_iota(jnp.int32, (L, L), 0)
    lj = lax.broadcasted_iota(jnp.int32, (L, L), 1)
    ltri = jnp.where(lj < li, 1.0, 0.0)
    lane1 = lax.broadcasted_iota(jnp.int32, (1, L), 1)

    def row(c, t):
        return box_ref[0, c, t][None]

    def block_body(state):
        t, cnt = state
        by1 = row(0, t)
        bx1 = row(1, t)
        by2 = row(2, t)
        bx2 = row(3, t)
        ba = (by2 - by1) * (bx2 - bx1)
        bval = jnp.where(t * L + lane1 < TOPK, 1.0, 0.0)
        packed = jnp.concatenate(
            [by1, bx1, by2, bx2, ba, bval, jnp.zeros((2, L), jnp.float32)],
            axis=0)
        tp = lax.transpose(packed, (1, 0))
        ci = [tp[:, i:i + 1] for i in range(5)]
        cval = tp[:, 5:6]

        def cross(s, acc):
            ry1 = row(0, s)
            rx1 = row(1, s)
            ry2 = row(2, s)
            rx2 = row(3, s)
            ra = (ry2 - ry1) * (rx2 - rx1)
            rk = keep_ref[0, s][None]
            m = _iou_gt(*ci, ry1, rx1, ry2, rx2, ra) & (rk > 0.5)
            return jnp.maximum(
                acc, jnp.max(jnp.where(m, 1.0, 0.0), axis=1, keepdims=True))

        sup0 = lax.fori_loop(0, t, cross, jnp.zeros((L, 1), jnp.float32))

        M = jnp.where(_iou_gt(*ci, by1, bx1, by2, bx2, ba), 1.0, 0.0) * ltri
        init = (1.0 - sup0) * cval

        def fp_cond(st):
            it, ch, _ = st
            return (ch > 0.5) & (it < L)

        def fp_body(st):
            it, _, kc = st
            sup = jnp.dot(M, kc, preferred_element_type=jnp.float32)
            nk = init * jnp.where(sup > 0.5, 0.0, 1.0)
            return it + 1, jnp.max(jnp.abs(nk - kc)), nk

        _, _, kc = lax.while_loop(fp_cond, fp_body, (0, 1.0, init))

        kr = lax.transpose(jnp.broadcast_to(kc, (L, 8)), (1, 0))[0]
        keep_ref[0, t] = kr
        return t + 1, cnt + jnp.sum(kc).astype(jnp.int32)

    def block_cond(state):
        t, cnt = state
        return (t < NBLK) & (cnt < NOUT)

    lax.while_loop(block_cond, block_body, (0, jnp.int32(0)))

    K = keep_ref[0]
    lt_inc = jnp.where(li <= lj, 1.0, 0.0)
    incl_row = jnp.dot(K, lt_inc, preferred_element_type=jnp.float32)
    rowsum = jnp.dot(K, jnp.ones((L, 1), jnp.float32),
                     preferred_element_type=jnp.float32)
    ri = lax.broadcasted_iota(jnp.int32, (NBLK, NBLK), 0)
    rj = lax.broadcasted_iota(jnp.int32, (NBLK, NBLK), 1)
    t48 = jnp.where(rj < ri, 1.0, 0.0)
    off = jnp.dot(t48, rowsum, preferred_element_type=jnp.float32)
    pos = (incl_row + off - 1.0).astype(jnp.int32)

    kept = K > 0.5
    zrows = jnp.zeros((64 - NBLK, L), jnp.float32)
    skey = jnp.concatenate(
        [jnp.where(kept, pos, BIG), jnp.full((64 - NBLK, L), BIG, jnp.int32)],
        axis=0)
    payload = [
        jnp.concatenate([jnp.where(kept, v, 0.0), zrows], axis=0)
        for v in (y1, x1, y2, x2)
    ]

    _, payload = _bitonic_keyed(skey, payload, 64, lambda k, pk: k < pk)
    for c in range(4):
        out_ref[0, c] = payload[c][:8]


def kernel(rpn_probs, rpn_bbox, anchors):
    nb = rpn_probs.shape[0]
    scores = rpn_probs[:, :, 1]
    sp = jnp.pad(scores, ((0, 0), (0, NROWS_IN * L - NPTS)),
                 constant_values=-1.0).reshape(nb, NROWS_IN, L)

    idx_sorted = pl.pallas_call(
        _sort_body,
        grid=(nb,),
        in_specs=[pl.BlockSpec((1, NROWS_IN, L), lambda b: (b, 0, 0))],
        out_specs=pl.BlockSpec((1, NBLK, L), lambda b: (b, 0, 0)),
        out_shape=jax.ShapeDtypeStruct((nb, NBLK, L), jnp.int32),
    )(sp)

    PPAD = 20096
    planes_src = jnp.concatenate(
        [anchors.transpose(0, 2, 1), rpn_bbox.transpose(0, 2, 1)], axis=1)
    flat = jnp.pad(planes_src, ((0, 0), (0, 0), (0, PPAD - NPTS))).reshape(-1)
    plane_off = ((jnp.arange(nb, dtype=jnp.int32) * 8)[:, None]
                 + jnp.arange(8, dtype=jnp.int32)[None, :]) * PPAD
    idx_g = (idx_sorted.reshape(nb, 1, CPAD)
             + plane_off[:, :, None]).reshape(nb * 8, NBLK, L)

    gathered = _sc_gather(flat, idx_g, nb * 8, NBLK)
    planes = gathered.reshape(nb, 8, NBLK, L)

    outp = pl.pallas_call(
        _nms_body,
        grid=(nb,),
        in_specs=[pl.BlockSpec((1, 8, NBLK, L), lambda b: (b, 0, 0, 0))],
        out_specs=pl.BlockSpec((1, 4, 8, L), lambda b: (b, 0, 0, 0)),
        out_shape=jax.ShapeDtypeStruct((nb, 4, 8, L), jnp.float32),
        scratch_shapes=[
            pltpu.VMEM((1, NBLK, L), jnp.float32),
            pltpu.VMEM((1, 4, NBLK, L), jnp.float32),
        ],
    )(planes)

    out = outp.reshape(nb, 4, 8 * L)[:, :, :NOUT]
    return out.transpose(0, 2, 1)
